# baseline (speedup 1.0000x reference)
import functools

import jax
import jax.numpy as jnp
from jax import lax
from jax.experimental import pallas as pl
from jax.experimental.pallas import tpu as pltpu
from jax.experimental.pallas import tpu_sc as plsc

NC = 2
NS = 16
LANES = 16
CH = 80

_SPLAT_DN = lax.GatherDimensionNumbers(
    offset_dims=(), collapsed_slice_dims=(0,), start_index_map=(0,))


def _lane_splat(v, rr):
    idx = jnp.full((LANES, 1), rr, jnp.int32)
    return lax.gather(v, idx, _SPLAT_DN, (1,),
                      mode=lax.GatherScatterMode.PROMISE_IN_BOUNDS)


def _pipelined(n, start, process):
    start(0, 0)

    def it(j, c):
        c1 = 2 * j + 1
        start(c1, 1)
        process(2 * j, 0)
        c2 = 2 * j + 2
        if n % 2 == 0:
            @pl.when(c2 < n)
            def _():
                start(c2, 0)
        else:
            start(c2, 0)
        process(c1, 1)
        return c
    lax.fori_loop(0, (n - 1 + 1) // 2 if n % 2 == 0 else (n - 1) // 2,
                  it, 0)
    if n % 2 == 1:
        process(n - 1, 0)


_SC_PARAMS = pltpu.CompilerParams(use_tc_tiling_on_sc=False,
                                  needs_layout_passes=False)
_SC_MESH = dict(core_axis_name="c", subcore_axis_name="s")


def _sc_count_kernel(n_nodes, n_edges):
    ew = n_edges // NS
    ch_per_w = ew // CH
    half = n_nodes * LANES // NC
    per_sub = half // 8
    mesh = plsc.VectorSubcoreMesh(**_SC_MESH)

    out_type = jax.ShapeDtypeStruct((NC, 8, per_sub), jnp.float32)

    scratch = [
        pltpu.VMEM((ew,), jnp.int32),
        pltpu.VMEM((ew,), jnp.int32),
        pltpu.VMEM((ch_per_w, CH), jnp.int32),
        pltpu.VMEM((CH,), jnp.float32),
        pltpu.VMEM((per_sub,), jnp.float32),
        pltpu.SemaphoreType.DMA,
        pltpu.VMEM_SHARED((n_nodes * LANES,), jnp.float32),
    ]

    @functools.partial(pl.kernel, mesh=mesh, out_type=out_type,
                       scratch_types=scratch, compiler_params=_SC_PARAMS)
    def kern(eidx_h, et_h, inv_h, dst_v, et_v, fidx_v, ones_v, inv_v,
             sem, acc):
        cid = lax.axis_index("c")
        sid = lax.axis_index("s")
        e0 = sid * ew

        pltpu.sync_copy(eidx_h.at[1, pl.ds(e0, ew)], dst_v)
        pltpu.sync_copy(et_h.at[pl.ds(e0, ew)], et_v)

        def fones(i, c):
            ones_v[pl.ds(i * LANES, LANES)] = jnp.full((LANES,), 1.0,
                                                       jnp.float32)
            return c
        lax.fori_loop(0, CH // LANES, fones, 0)

        def frow(r, c):
            for i in range(CH // LANES):
                s = pl.ds(r * CH + i * LANES, LANES)
                fidx_v[r, pl.ds(i * LANES, LANES)] = (
                    dst_v[s] * LANES + et_v[s])
            return c
        lax.fori_loop(0, ch_per_w, frow, 0, unroll=4)

        zw = n_nodes * LANES // NS
        def zrow(i, c):
            inv_v[pl.ds(i * LANES, LANES)] = jnp.zeros((LANES,),
                                                       jnp.float32)
            return c
        lax.fori_loop(0, zw // LANES, zrow, 0, unroll=8)
        pltpu.sync_copy(inv_v.at[pl.ds(0, zw)],
                        acc.at[pl.ds(sid * zw, zw)])

        plsc.subcore_barrier()

        def scount(g, c):
            for k in range(10):
                pltpu.async_copy(ones_v, acc.at[fidx_v.at[g * 10 + k]],
                                 sem, add=True)
            for k in range(10):
                pltpu.make_async_copy(ones_v, acc.at[fidx_v.at[0]],
                                      sem).wait()
            return c
        lax.fori_loop(0, ch_per_w // 10, scount, 0)

        plsc.subcore_barrier()

        @pl.when(sid < 8)
        def _():
            off = cid * half + sid * per_sub
            pltpu.sync_copy(acc.at[pl.ds(off, per_sub)], inv_v)

            def irow(i, c):
                s = pl.ds(i * LANES, LANES)
                inv_v[s] = 1.0 / jnp.maximum(inv_v[s], 1.0)
                return c
            lax.fori_loop(0, per_sub // LANES, irow, 0, unroll=8)
            pltpu.sync_copy(inv_v, inv_h.at[cid, sid])

    return kern


def _sc_message_kernel(mode, n_nodes, n_edges):
    nw = NC * NS
    ew = n_edges // nw
    ch_per_w = ew // CH
    mesh = plsc.VectorSubcoreMesh(**_SC_MESH)

    out_type = [jax.ShapeDtypeStruct((NC, n_nodes, LANES), jnp.float32)]
    if mode == "first":
        out_type.append(jax.ShapeDtypeStruct((nw, ew), jnp.float32))

    NB = 2

    scratch = (
        [pltpu.VMEM((ew,), jnp.int32),
         pltpu.VMEM((ew,), jnp.int32),
         pltpu.VMEM((ew,), jnp.int32),
         pltpu.VMEM((ch_per_w, CH), jnp.int32),
         pltpu.VMEM((ch_per_w, CH), jnp.int32),
         pltpu.VMEM((ew,), jnp.float32),
         pltpu.VMEM((n_nodes // NS, LANES), jnp.float32)]
        + [pltpu.VMEM((CH, LANES), jnp.float32)] * NB
        + [pltpu.VMEM((CH, LANES), jnp.float32)] * NB
        + [pltpu.SemaphoreType.DMA] * (3 * NB)
        + [pltpu.VMEM_SHARED((n_nodes, LANES), jnp.float32)]
    )

    @functools.partial(pl.kernel, mesh=mesh, out_type=tuple(out_type),
                       scratch_types=scratch, compiler_params=_SC_PARAMS)
    def kern(*refs):
        if mode == "first":
            (table_h, eidx_h, et_h, inv_h, out_h, w_out_h, *rest) = refs
        else:
            (table_h, eidx_h, et_h, w_h, out_h, *rest) = refs
        (src_v, dst_v, et_v, gidx_v, sidx_v, w_v, zbuf, *rest) = rest
        bm = rest[:NB]
        bi = rest[NB:2 * NB]
        sems = rest[2 * NB:5 * NB]
        gs = sems[:NB]
        isems = sems[NB:2 * NB]
        ss = sems[2 * NB:]
        acc = rest[5 * NB]

        cid = lax.axis_index("c")
        sid = lax.axis_index("s")
        wid = sid * NC + cid
        e0 = wid * ew

        pltpu.sync_copy(eidx_h.at[0, pl.ds(e0, ew)], src_v)
        pltpu.sync_copy(eidx_h.at[1, pl.ds(e0, ew)], dst_v)
        pltpu.sync_copy(et_h.at[pl.ds(e0, ew)], et_v)
        if mode == "second":
            pltpu.sync_copy(w_h.at[wid], w_v)

        def frow(r, c):
            for i in range(CH // LANES):
                s = pl.ds(r * CH + i * LANES, LANES)
                d = pl.ds(i * LANES, LANES)
                gidx_v[r, d] = src_v[s] * LANES + et_v[s]
                sidx_v[r, d] = dst_v[s]
            return c
        lax.fori_loop(0, ch_per_w, frow, 0, unroll=4)

        nslice = n_nodes // NS
        def zrow(i, c):
            zbuf[i, :] = jnp.zeros((LANES,), jnp.float32)
            return c
        lax.fori_loop(0, nslice, zrow, 0, unroll=8)
        pltpu.sync_copy(zbuf, acc.at[pl.ds(sid * nslice, nslice)])

        plsc.subcore_barrier()

        n = ch_per_w

        def gstart(c, k):
            pltpu.async_copy(table_h.at[gidx_v.at[c]], bm[k], gs[k])
            if mode == "first":
                pltpu.async_copy(inv_h.at[sidx_v.at[c]], bi[k], isems[k])

        def swait(k):
            pltpu.make_async_copy(bm[k], acc.at[sidx_v.at[0]],
                                  ss[k]).wait()

        def step(c, k):
            pltpu.make_async_copy(table_h.at[gidx_v.at[0]], bm[k],
                                  gs[k]).wait()
            if mode == "first":
                pltpu.make_async_copy(inv_h.at[sidx_v.at[0]], bi[k],
                                      isems[k]).wait()
            for i in range(CH // LANES):
                if mode == "first":
                    ridx = lax.iota(jnp.int32, LANES) + i * LANES
                    lidx = et_v[pl.ds(c * CH + i * LANES, LANES)]
                    wv = plsc.load_gather(bi[k], [ridx, lidx])
                    w_v[pl.ds(c * CH + i * LANES, LANES)] = wv
                else:
                    wv = w_v[pl.ds(c * CH + i * LANES, LANES)]
                for rr in range(LANES):
                    r = i * LANES + rr
                    bm[k][r, :] = bm[k][r, :] * _lane_splat(wv, rr)
            pltpu.async_copy(bm[k], acc.at[sidx_v.at[c]], ss[k], add=True)
            kn = (k + 1) % NB
            @pl.when(c >= NB - 1)
            def _():
                swait(kn)

            @pl.when(c + 1 < n)
            def _():
                gstart(c + 1, kn)

        gstart(0, 0)

        def trio(m, carry):
            for k in range(NB):
                c = m * NB + k

                @pl.when(c < n)
                def _():
                    step(c, k)
            return carry
        lax.fori_loop(0, (n + NB - 1) // NB, trio, 0)
        for t in range(NB - 1):
            swait((n - NB + 1 + t) % NB)

        if mode == "first":
            pltpu.sync_copy(w_v, w_out_h.at[wid])

        plsc.subcore_barrier()

        @pl.when(sid == 0)
        def _():
            pltpu.sync_copy(acc, out_h.at[cid])

    return kern


def _tc_layer1(x_ref, w1_ref, r1_ref, b1_ref, xr_ref, s1_ref):
    x = x_ref[...]
    xr_ref[...] = jnp.dot(x, w1_ref[...], preferred_element_type=jnp.float32)
    s1_ref[...] = (jnp.dot(x, r1_ref[...],
                           preferred_element_type=jnp.float32) + b1_ref[...])


def _tc_layer2(s1_ref, acc_ref, w2_ref, r2_ref, b2_ref, hr_ref, s2_ref):
    h1 = jnp.maximum(s1_ref[...] + acc_ref[0] + acc_ref[1], 0.0)
    hr_ref[...] = jnp.dot(h1, w2_ref[...], preferred_element_type=jnp.float32)
    s2_ref[...] = (jnp.dot(h1, r2_ref[...],
                           preferred_element_type=jnp.float32) + b2_ref[...])


def _tc_head(s2_ref, acc_ref, batch_ref, wfc_ref, bfc_ref, out_ref, *, G):
    h2 = jnp.maximum(s2_ref[...] + acc_ref[0] + acc_ref[1], 0.0)
    n = h2.shape[0]
    gids = lax.broadcasted_iota(jnp.int32, (1, G), 1)
    bm = (batch_ref[...] == gids).astype(jnp.float32)
    pooled = lax.dot_general(bm, h2, (((0,), (0,)), ((), ())),
                             preferred_element_type=jnp.float32)
    cnt = lax.dot_general(bm, jnp.ones((n, 1), jnp.float32),
                          (((0,), (0,)), ((), ())),
                          preferred_element_type=jnp.float32)
    g = pooled / jnp.maximum(cnt, 1.0)
    out_ref[...] = jnp.dot(g, wfc_ref[...],
                           preferred_element_type=jnp.float32) + bfc_ref[...]


def kernel(x, edge_index, edge_type, batch, W1, root1, b1, W2, root2, b2,
           Wfc, bfc):
    N, F_IN = x.shape
    E = edge_index.shape[1]
    R, _, H = W1.shape
    G = 64
    C = Wfc.shape[1]

    eidx = edge_index.astype(jnp.int32)
    et = edge_type.astype(jnp.int32)

    W1cat = W1.transpose(1, 0, 2).reshape(F_IN, R * H)
    W2cat = W2.transpose(1, 0, 2).reshape(H, R * H)

    inv = _sc_count_kernel(N, E)(eidx, et)
    if isinstance(inv, (tuple, list)):
        inv = inv[0]
    inv = inv.reshape(N, LANES)

    xr, self1 = pl.pallas_call(
        _tc_layer1,
        out_shape=[
            jax.ShapeDtypeStruct((N, R * H), jnp.float32),
            jax.ShapeDtypeStruct((N, H), jnp.float32),
        ],
    )(x, W1cat, root1, b1.reshape(1, H))

    acc1, w_e = _sc_message_kernel("first", N, E)(
        xr.reshape(N * R, H), eidx, et, inv)

    hr2, self2 = pl.pallas_call(
        _tc_layer2,
        out_shape=[
            jax.ShapeDtypeStruct((N, R * H), jnp.float32),
            jax.ShapeDtypeStruct((N, H), jnp.float32),
        ],
    )(self1, acc1, W2cat, root2, b2.reshape(1, H))

    acc2 = _sc_message_kernel("second", N, E)(
        hr2.reshape(N * R, H), eidx, et, w_e)
    if isinstance(acc2, (tuple, list)):
        acc2 = acc2[0]

    out = pl.pallas_call(
        functools.partial(_tc_head, G=G),
        out_shape=jax.ShapeDtypeStruct((G, C), jnp.float32),
    )(self2, acc2, batch.reshape(N, 1), Wfc, bfc.reshape(1, C))

    return out


# speedup vs baseline: 1.0848x; 1.0848x over previous
import functools

import jax
import jax.numpy as jnp
from jax import lax
from jax.experimental import pallas as pl
from jax.experimental.pallas import tpu as pltpu
from jax.experimental.pallas import tpu_sc as plsc

NC = 2
NS = 16
LANES = 16
CH = 80

_SPLAT_DN = lax.GatherDimensionNumbers(
    offset_dims=(), collapsed_slice_dims=(0,), start_index_map=(0,))


def _lane_splat(v, rr):
    idx = jnp.full((LANES, 1), rr, jnp.int32)
    return lax.gather(v, idx, _SPLAT_DN, (1,),
                      mode=lax.GatherScatterMode.PROMISE_IN_BOUNDS)


def _pipelined(n, start, process):
    start(0, 0)

    def it(j, c):
        c1 = 2 * j + 1
        start(c1, 1)
        process(2 * j, 0)
        c2 = 2 * j + 2
        if n % 2 == 0:
            @pl.when(c2 < n)
            def _():
                start(c2, 0)
        else:
            start(c2, 0)
        process(c1, 1)
        return c
    lax.fori_loop(0, (n - 1 + 1) // 2 if n % 2 == 0 else (n - 1) // 2,
                  it, 0)
    if n % 2 == 1:
        process(n - 1, 0)


_SC_PARAMS = pltpu.CompilerParams(use_tc_tiling_on_sc=False,
                                  needs_layout_passes=False)
_SC_MESH = dict(core_axis_name="c", subcore_axis_name="s")


def _sc_count_kernel(n_nodes, n_edges):
    ew = n_edges // NS
    ch_per_w = ew // CH
    half = n_nodes * LANES // NC
    per_sub = half // 8
    mesh = plsc.VectorSubcoreMesh(**_SC_MESH)

    out_type = jax.ShapeDtypeStruct((NC, 8, per_sub), jnp.float32)

    scratch = [
        pltpu.VMEM((ew,), jnp.int32),
        pltpu.VMEM((ew,), jnp.int32),
        pltpu.VMEM((ch_per_w, CH), jnp.int32),
        pltpu.VMEM((CH,), jnp.float32),
        pltpu.VMEM((per_sub,), jnp.float32),
        pltpu.SemaphoreType.DMA,
        pltpu.VMEM_SHARED((n_nodes * LANES,), jnp.float32),
    ]

    @functools.partial(pl.kernel, mesh=mesh, out_type=out_type,
                       scratch_types=scratch, compiler_params=_SC_PARAMS)
    def kern(eidx_h, et_h, inv_h, dst_v, et_v, fidx_v, ones_v, inv_v,
             sem, acc):
        cid = lax.axis_index("c")
        sid = lax.axis_index("s")
        e0 = sid * ew

        pltpu.sync_copy(eidx_h.at[1, pl.ds(e0, ew)], dst_v)
        pltpu.sync_copy(et_h.at[pl.ds(e0, ew)], et_v)

        def fones(i, c):
            ones_v[pl.ds(i * LANES, LANES)] = jnp.full((LANES,), 1.0,
                                                       jnp.float32)
            return c
        lax.fori_loop(0, CH // LANES, fones, 0)

        def frow(r, c):
            for i in range(CH // LANES):
                s = pl.ds(r * CH + i * LANES, LANES)
                fidx_v[r, pl.ds(i * LANES, LANES)] = (
                    dst_v[s] * LANES + et_v[s])
            return c
        lax.fori_loop(0, ch_per_w, frow, 0, unroll=4)

        zw = n_nodes * LANES // NS
        def zrow(i, c):
            inv_v[pl.ds(i * LANES, LANES)] = jnp.zeros((LANES,),
                                                       jnp.float32)
            return c
        lax.fori_loop(0, zw // LANES, zrow, 0, unroll=8)
        pltpu.sync_copy(inv_v.at[pl.ds(0, zw)],
                        acc.at[pl.ds(sid * zw, zw)])

        plsc.subcore_barrier()

        def scount(g, c):
            for k in range(10):
                pltpu.async_copy(ones_v, acc.at[fidx_v.at[g * 10 + k]],
                                 sem, add=True)
            for k in range(10):
                pltpu.make_async_copy(ones_v, acc.at[fidx_v.at[0]],
                                      sem).wait()
            return c
        lax.fori_loop(0, ch_per_w // 10, scount, 0)

        plsc.subcore_barrier()

        @pl.when(sid < 8)
        def _():
            off = cid * half + sid * per_sub
            pltpu.sync_copy(acc.at[pl.ds(off, per_sub)], inv_v)

            def irow(i, c):
                s = pl.ds(i * LANES, LANES)
                inv_v[s] = 1.0 / jnp.maximum(inv_v[s], 1.0)
                return c
            lax.fori_loop(0, per_sub // LANES, irow, 0, unroll=8)
            pltpu.sync_copy(inv_v, inv_h.at[cid, sid])

    return kern


def _sc_message_kernel(mode, n_nodes, n_edges):
    nw = NC * NS
    ew = n_edges // nw
    ch_per_w = ew // CH
    mesh = plsc.VectorSubcoreMesh(**_SC_MESH)

    out_type = [jax.ShapeDtypeStruct((NC, n_nodes, LANES), jnp.float32)]
    if mode == "first":
        out_type.append(jax.ShapeDtypeStruct((nw, ew), jnp.float32))

    NB = 3

    scratch = (
        [pltpu.VMEM((ew,), jnp.int32),
         pltpu.VMEM((ew,), jnp.int32),
         pltpu.VMEM((ew,), jnp.int32),
         pltpu.VMEM((ch_per_w, CH), jnp.int32),
         pltpu.VMEM((ch_per_w, CH), jnp.int32),
         pltpu.VMEM((ew,), jnp.float32),
         pltpu.VMEM((n_nodes // NS, LANES), jnp.float32)]
        + [pltpu.VMEM((CH, LANES), jnp.float32)] * NB
        + [pltpu.VMEM((CH, LANES), jnp.float32)] * NB
        + [pltpu.SemaphoreType.DMA] * (3 * NB)
        + [pltpu.VMEM_SHARED((n_nodes, LANES), jnp.float32)]
    )

    @functools.partial(pl.kernel, mesh=mesh, out_type=tuple(out_type),
                       scratch_types=scratch, compiler_params=_SC_PARAMS)
    def kern(*refs):
        if mode == "first":
            (table_h, eidx_h, et_h, inv_h, out_h, w_out_h, *rest) = refs
        else:
            (table_h, eidx_h, et_h, w_h, out_h, *rest) = refs
        (src_v, dst_v, et_v, gidx_v, sidx_v, w_v, zbuf, *rest) = rest
        bm = rest[:NB]
        bi = rest[NB:2 * NB]
        sems = rest[2 * NB:5 * NB]
        gs = sems[:NB]
        isems = sems[NB:2 * NB]
        ss = sems[2 * NB:]
        acc = rest[5 * NB]

        cid = lax.axis_index("c")
        sid = lax.axis_index("s")
        wid = sid * NC + cid
        e0 = wid * ew

        pltpu.sync_copy(eidx_h.at[0, pl.ds(e0, ew)], src_v)
        pltpu.sync_copy(eidx_h.at[1, pl.ds(e0, ew)], dst_v)
        pltpu.sync_copy(et_h.at[pl.ds(e0, ew)], et_v)
        if mode == "second":
            pltpu.sync_copy(w_h.at[wid], w_v)

        def frow(r, c):
            for i in range(CH // LANES):
                s = pl.ds(r * CH + i * LANES, LANES)
                d = pl.ds(i * LANES, LANES)
                gidx_v[r, d] = src_v[s] * LANES + et_v[s]
                sidx_v[r, d] = dst_v[s]
            return c
        lax.fori_loop(0, ch_per_w, frow, 0, unroll=4)

        nslice = n_nodes // NS
        def zrow(i, c):
            zbuf[i, :] = jnp.zeros((LANES,), jnp.float32)
            return c
        lax.fori_loop(0, nslice, zrow, 0, unroll=8)
        pltpu.sync_copy(zbuf, acc.at[pl.ds(sid * nslice, nslice)])

        plsc.subcore_barrier()

        n = ch_per_w

        def gstart(c, k):
            pltpu.async_copy(table_h.at[gidx_v.at[c]], bm[k], gs[k])
            if mode == "first":
                pltpu.async_copy(inv_h.at[sidx_v.at[c]], bi[k], isems[k])

        def swait(k):
            pltpu.make_async_copy(bm[k], acc.at[sidx_v.at[0]],
                                  ss[k]).wait()

        def step(c, k):
            pltpu.make_async_copy(table_h.at[gidx_v.at[0]], bm[k],
                                  gs[k]).wait()
            if mode == "first":
                pltpu.make_async_copy(inv_h.at[sidx_v.at[0]], bi[k],
                                      isems[k]).wait()
            kn = (k + 1) % NB
            @pl.when(c >= NB - 1)
            def _():
                swait(kn)

            @pl.when(c + 1 < n)
            def _():
                gstart(c + 1, kn)

            for i in range(CH // LANES):
                if mode == "first":
                    ridx = lax.iota(jnp.int32, LANES) + i * LANES
                    lidx = et_v[pl.ds(c * CH + i * LANES, LANES)]
                    wv = plsc.load_gather(bi[k], [ridx, lidx])
                    w_v[pl.ds(c * CH + i * LANES, LANES)] = wv
                else:
                    wv = w_v[pl.ds(c * CH + i * LANES, LANES)]
                for rr in range(LANES):
                    r = i * LANES + rr
                    bm[k][r, :] = bm[k][r, :] * _lane_splat(wv, rr)
            pltpu.async_copy(bm[k], acc.at[sidx_v.at[c]], ss[k], add=True)

        gstart(0, 0)

        def trio(m, carry):
            for k in range(NB):
                c = m * NB + k

                @pl.when(c < n)
                def _():
                    step(c, k)
            return carry
        lax.fori_loop(0, (n + NB - 1) // NB, trio, 0)
        for t in range(NB - 1):
            swait((n - NB + 1 + t) % NB)

        if mode == "first":
            pltpu.sync_copy(w_v, w_out_h.at[wid])

        plsc.subcore_barrier()

        @pl.when(sid == 0)
        def _():
            pltpu.sync_copy(acc, out_h.at[cid])

    return kern


def _tc_layer1(x_ref, w1_ref, r1_ref, b1_ref, xr_ref, s1_ref):
    x = x_ref[...]
    xr_ref[...] = jnp.dot(x, w1_ref[...], preferred_element_type=jnp.float32)
    s1_ref[...] = (jnp.dot(x, r1_ref[...],
                           preferred_element_type=jnp.float32) + b1_ref[...])


def _tc_layer2(s1_ref, acc_ref, w2_ref, r2_ref, b2_ref, hr_ref, s2_ref):
    h1 = jnp.maximum(s1_ref[...] + acc_ref[0] + acc_ref[1], 0.0)
    hr_ref[...] = jnp.dot(h1, w2_ref[...], preferred_element_type=jnp.float32)
    s2_ref[...] = (jnp.dot(h1, r2_ref[...],
                           preferred_element_type=jnp.float32) + b2_ref[...])


def _tc_head(s2_ref, acc_ref, batch_ref, wfc_ref, bfc_ref, out_ref, *, G):
    h2 = jnp.maximum(s2_ref[...] + acc_ref[0] + acc_ref[1], 0.0)
    n = h2.shape[0]
    gids = lax.broadcasted_iota(jnp.int32, (1, G), 1)
    bm = (batch_ref[...] == gids).astype(jnp.float32)
    pooled = lax.dot_general(bm, h2, (((0,), (0,)), ((), ())),
                             preferred_element_type=jnp.float32)
    cnt = lax.dot_general(bm, jnp.ones((n, 1), jnp.float32),
                          (((0,), (0,)), ((), ())),
                          preferred_element_type=jnp.float32)
    g = pooled / jnp.maximum(cnt, 1.0)
    out_ref[...] = jnp.dot(g, wfc_ref[...],
                           preferred_element_type=jnp.float32) + bfc_ref[...]


def kernel(x, edge_index, edge_type, batch, W1, root1, b1, W2, root2, b2,
           Wfc, bfc):
    N, F_IN = x.shape
    E = edge_index.shape[1]
    R, _, H = W1.shape
    G = 64
    C = Wfc.shape[1]

    eidx = edge_index.astype(jnp.int32)
    et = edge_type.astype(jnp.int32)

    W1cat = W1.transpose(1, 0, 2).reshape(F_IN, R * H)
    W2cat = W2.transpose(1, 0, 2).reshape(H, R * H)

    inv = _sc_count_kernel(N, E)(eidx, et)
    if isinstance(inv, (tuple, list)):
        inv = inv[0]
    inv = inv.reshape(N, LANES)

    xr, self1 = pl.pallas_call(
        _tc_layer1,
        out_shape=[
            jax.ShapeDtypeStruct((N, R * H), jnp.float32),
            jax.ShapeDtypeStruct((N, H), jnp.float32),
        ],
    )(x, W1cat, root1, b1.reshape(1, H))

    acc1, w_e = _sc_message_kernel("first", N, E)(
        xr.reshape(N * R, H), eidx, et, inv)

    hr2, self2 = pl.pallas_call(
        _tc_layer2,
        out_shape=[
            jax.ShapeDtypeStruct((N, R * H), jnp.float32),
            jax.ShapeDtypeStruct((N, H), jnp.float32),
        ],
    )(self1, acc1, W2cat, root2, b2.reshape(1, H))

    acc2 = _sc_message_kernel("second", N, E)(
        hr2.reshape(N * R, H), eidx, et, w_e)
    if isinstance(acc2, (tuple, list)):
        acc2 = acc2[0]

    out = pl.pallas_call(
        functools.partial(_tc_head, G=G),
        out_shape=jax.ShapeDtypeStruct((G, C), jnp.float32),
    )(self2, acc2, batch.reshape(N, 1), Wfc, bfc.reshape(1, C))

    return out


# speedup vs baseline: 1.3269x; 1.2232x over previous
import functools

import jax
import jax.numpy as jnp
from jax import lax
from jax.experimental import pallas as pl
from jax.experimental.pallas import tpu as pltpu
from jax.experimental.pallas import tpu_sc as plsc

NC = 2
NS = 16
LANES = 16
CH = 80

_SPLAT_DN = lax.GatherDimensionNumbers(
    offset_dims=(), collapsed_slice_dims=(0,), start_index_map=(0,))


def _lane_splat(v, rr):
    idx = jnp.full((LANES, 1), rr, jnp.int32)
    return lax.gather(v, idx, _SPLAT_DN, (1,),
                      mode=lax.GatherScatterMode.PROMISE_IN_BOUNDS)


def _pipelined(n, start, process):
    start(0, 0)

    def it(j, c):
        c1 = 2 * j + 1
        start(c1, 1)
        process(2 * j, 0)
        c2 = 2 * j + 2
        if n % 2 == 0:
            @pl.when(c2 < n)
            def _():
                start(c2, 0)
        else:
            start(c2, 0)
        process(c1, 1)
        return c
    lax.fori_loop(0, (n - 1 + 1) // 2 if n % 2 == 0 else (n - 1) // 2,
                  it, 0)
    if n % 2 == 1:
        process(n - 1, 0)


_SC_PARAMS = pltpu.CompilerParams(use_tc_tiling_on_sc=False,
                                  needs_layout_passes=False)
_SC_MESH = dict(core_axis_name="c", subcore_axis_name="s")


def _sc_count_kernel(n_nodes, n_edges):
    ew = n_edges // NS
    ch_per_w = ew // CH
    half = n_nodes * LANES // NC
    per_sub = half // 8
    mesh = plsc.VectorSubcoreMesh(**_SC_MESH)

    out_type = jax.ShapeDtypeStruct((NC, 8, per_sub), jnp.float32)

    scratch = [
        pltpu.VMEM((ew,), jnp.int32),
        pltpu.VMEM((ew,), jnp.int32),
        pltpu.VMEM((ch_per_w, CH), jnp.int32),
        pltpu.VMEM((CH,), jnp.float32),
        pltpu.VMEM((per_sub,), jnp.float32),
        pltpu.SemaphoreType.DMA,
        pltpu.VMEM_SHARED((n_nodes * LANES,), jnp.float32),
    ]

    @functools.partial(pl.kernel, mesh=mesh, out_type=out_type,
                       scratch_types=scratch, compiler_params=_SC_PARAMS)
    def kern(eidx_h, et_h, inv_h, dst_v, et_v, fidx_v, ones_v, inv_v,
             sem, acc):
        cid = lax.axis_index("c")
        sid = lax.axis_index("s")
        e0 = sid * ew

        pltpu.sync_copy(eidx_h.at[1, pl.ds(e0, ew)], dst_v)
        pltpu.sync_copy(et_h.at[pl.ds(e0, ew)], et_v)

        def fones(i, c):
            ones_v[pl.ds(i * LANES, LANES)] = jnp.full((LANES,), 1.0,
                                                       jnp.float32)
            return c
        lax.fori_loop(0, CH // LANES, fones, 0)

        def frow(r, c):
            for i in range(CH // LANES):
                s = pl.ds(r * CH + i * LANES, LANES)
                fidx_v[r, pl.ds(i * LANES, LANES)] = (
                    dst_v[s] * LANES + et_v[s])
            return c
        lax.fori_loop(0, ch_per_w, frow, 0, unroll=4)

        zw = n_nodes * LANES // NS
        def zrow(i, c):
            inv_v[pl.ds(i * LANES, LANES)] = jnp.zeros((LANES,),
                                                       jnp.float32)
            return c
        lax.fori_loop(0, zw // LANES, zrow, 0, unroll=8)
        pltpu.sync_copy(inv_v.at[pl.ds(0, zw)],
                        acc.at[pl.ds(sid * zw, zw)])

        plsc.subcore_barrier()

        def scount(g, c):
            for k in range(10):
                pltpu.async_copy(ones_v, acc.at[fidx_v.at[g * 10 + k]],
                                 sem, add=True)
            for k in range(10):
                pltpu.make_async_copy(ones_v, acc.at[fidx_v.at[0]],
                                      sem).wait()
            return c
        lax.fori_loop(0, ch_per_w // 10, scount, 0)

        plsc.subcore_barrier()

        @pl.when(sid < 8)
        def _():
            off = cid * half + sid * per_sub
            pltpu.sync_copy(acc.at[pl.ds(off, per_sub)], inv_v)

            def irow(i, c):
                s = pl.ds(i * LANES, LANES)
                inv_v[s] = 1.0 / jnp.maximum(inv_v[s], 1.0)
                return c
            lax.fori_loop(0, per_sub // LANES, irow, 0, unroll=8)
            pltpu.sync_copy(inv_v, inv_h.at[cid, sid])

    return kern


def _sc_message_kernel(mode, n_nodes, n_edges):
    nw = NC * NS
    ew = n_edges // nw
    ch_per_w = ew // CH
    mesh = plsc.VectorSubcoreMesh(**_SC_MESH)

    out_type = [jax.ShapeDtypeStruct((NC, n_nodes, LANES), jnp.float32)]
    if mode == "first":
        out_type.append(jax.ShapeDtypeStruct((nw, ew), jnp.float32))

    NB = 2

    scratch = (
        [pltpu.VMEM((ew,), jnp.int32),
         pltpu.VMEM((ew,), jnp.int32),
         pltpu.VMEM((ew,), jnp.int32),
         pltpu.VMEM((ch_per_w, CH), jnp.int32),
         pltpu.VMEM((ch_per_w, CH), jnp.int32),
         pltpu.VMEM((ew,), jnp.float32),
         pltpu.VMEM((n_nodes // NS, LANES), jnp.float32)]
        + [pltpu.VMEM((CH, LANES), jnp.float32)] * NB
        + [pltpu.VMEM((CH, LANES), jnp.float32)] * NB
        + [pltpu.SemaphoreType.DMA] * (3 * NB)
        + [pltpu.VMEM_SHARED((n_nodes, LANES), jnp.float32)]
    )

    @functools.partial(pl.kernel, mesh=mesh, out_type=tuple(out_type),
                       scratch_types=scratch, compiler_params=_SC_PARAMS)
    def kern(*refs):
        if mode == "first":
            (table_h, eidx_h, et_h, inv_h, out_h, w_out_h, *rest) = refs
        else:
            (table_h, eidx_h, et_h, w_h, out_h, *rest) = refs
        (src_v, dst_v, et_v, gidx_v, sidx_v, w_v, zbuf, *rest) = rest
        bm = rest[:NB]
        bi = rest[NB:2 * NB]
        sems = rest[2 * NB:5 * NB]
        gs = sems[:NB]
        isems = sems[NB:2 * NB]
        ss = sems[2 * NB:]
        acc = rest[5 * NB]

        cid = lax.axis_index("c")
        sid = lax.axis_index("s")
        wid = sid * NC + cid
        e0 = wid * ew

        pltpu.sync_copy(eidx_h.at[0, pl.ds(e0, ew)], src_v)
        pltpu.sync_copy(eidx_h.at[1, pl.ds(e0, ew)], dst_v)
        pltpu.sync_copy(et_h.at[pl.ds(e0, ew)], et_v)
        if mode == "second":
            pltpu.sync_copy(w_h.at[wid], w_v)

        def frow(r, c):
            for i in range(CH // LANES):
                s = pl.ds(r * CH + i * LANES, LANES)
                d = pl.ds(i * LANES, LANES)
                gidx_v[r, d] = src_v[s] * LANES + et_v[s]
                sidx_v[r, d] = dst_v[s]
            return c
        lax.fori_loop(0, ch_per_w, frow, 0, unroll=4)

        nslice = n_nodes // NS
        def zrow(i, c):
            zbuf[i, :] = jnp.zeros((LANES,), jnp.float32)
            return c
        lax.fori_loop(0, nslice, zrow, 0, unroll=8)
        pltpu.sync_copy(zbuf, acc.at[pl.ds(sid * nslice, nslice)])

        plsc.subcore_barrier()

        def start(c, b):
            pltpu.async_copy(table_h.at[gidx_v.at[c]], bm[b], gs[b])
            if mode == "first":
                pltpu.async_copy(inv_h.at[sidx_v.at[c]], bi[b], isems[b])

        def process(c, b):
            pltpu.make_async_copy(table_h.at[gidx_v.at[0]], bm[b],
                                  gs[b]).wait()
            if mode == "first":
                pltpu.make_async_copy(inv_h.at[sidx_v.at[0]], bi[b],
                                      isems[b]).wait()
            for i in range(CH // LANES):
                if mode == "first":
                    ridx = lax.iota(jnp.int32, LANES) + i * LANES
                    lidx = et_v[pl.ds(c * CH + i * LANES, LANES)]
                    wv = plsc.load_gather(bi[b], [ridx, lidx])
                    w_v[pl.ds(c * CH + i * LANES, LANES)] = wv
                else:
                    wv = w_v[pl.ds(c * CH + i * LANES, LANES)]
                for rr in range(LANES):
                    r = i * LANES + rr
                    bm[b][r, :] = bm[b][r, :] * _lane_splat(wv, rr)
            pltpu.sync_copy(bm[b], acc.at[sidx_v.at[c]], add=True)

        _pipelined(ch_per_w, start, process)

        if mode == "first":
            pltpu.sync_copy(w_v, w_out_h.at[wid])

        plsc.subcore_barrier()

        @pl.when(sid == 0)
        def _():
            pltpu.sync_copy(acc, out_h.at[cid])

    return kern


def _tc_layer1(x_ref, w1_ref, r1_ref, b1_ref, xr_ref, s1_ref):
    x = x_ref[...]
    xr_ref[...] = jnp.dot(x, w1_ref[...], preferred_element_type=jnp.float32)
    s1_ref[...] = (jnp.dot(x, r1_ref[...],
                           preferred_element_type=jnp.float32) + b1_ref[...])


def _tc_layer2(s1_ref, acc_ref, w2_ref, r2_ref, b2_ref, hr_ref, s2_ref):
    h1 = jnp.maximum(s1_ref[...] + acc_ref[0] + acc_ref[1], 0.0)
    hr_ref[...] = jnp.dot(h1, w2_ref[...], preferred_element_type=jnp.float32)
    s2_ref[...] = (jnp.dot(h1, r2_ref[...],
                           preferred_element_type=jnp.float32) + b2_ref[...])


def _tc_head(s2_ref, acc_ref, batch_ref, wfc_ref, bfc_ref, out_ref, *, G):
    h2 = jnp.maximum(s2_ref[...] + acc_ref[0] + acc_ref[1], 0.0)
    n = h2.shape[0]
    gids = lax.broadcasted_iota(jnp.int32, (1, G), 1)
    bm = (batch_ref[...] == gids).astype(jnp.float32)
    pooled = lax.dot_general(bm, h2, (((0,), (0,)), ((), ())),
                             preferred_element_type=jnp.float32)
    cnt = lax.dot_general(bm, jnp.ones((n, 1), jnp.float32),
                          (((0,), (0,)), ((), ())),
                          preferred_element_type=jnp.float32)
    g = pooled / jnp.maximum(cnt, 1.0)
    out_ref[...] = jnp.dot(g, wfc_ref[...],
                           preferred_element_type=jnp.float32) + bfc_ref[...]


def kernel(x, edge_index, edge_type, batch, W1, root1, b1, W2, root2, b2,
           Wfc, bfc):
    N, F_IN = x.shape
    E = edge_index.shape[1]
    R, _, H = W1.shape
    G = 64
    C = Wfc.shape[1]

    eidx = edge_index.astype(jnp.int32)
    et = edge_type.astype(jnp.int32)

    W1cat = W1.transpose(1, 0, 2).reshape(F_IN, R * H)
    W2cat = W2.transpose(1, 0, 2).reshape(H, R * H)

    inv = _sc_count_kernel(N, E)(eidx, et)
    if isinstance(inv, (tuple, list)):
        inv = inv[0]
    inv = inv.reshape(N, LANES)

    xr, self1 = pl.pallas_call(
        _tc_layer1,
        out_shape=[
            jax.ShapeDtypeStruct((N, R * H), jnp.float32),
            jax.ShapeDtypeStruct((N, H), jnp.float32),
        ],
    )(x, W1cat, root1, b1.reshape(1, H))

    acc1, w_e = _sc_message_kernel("first", N, E)(
        xr.reshape(N * R, H), eidx, et, inv)

    hr2, self2 = pl.pallas_call(
        _tc_layer2,
        out_shape=[
            jax.ShapeDtypeStruct((N, R * H), jnp.float32),
            jax.ShapeDtypeStruct((N, H), jnp.float32),
        ],
    )(self1, acc1, W2cat, root2, b2.reshape(1, H))

    acc2 = _sc_message_kernel("second", N, E)(
        hr2.reshape(N * R, H), eidx, et, w_e)
    if isinstance(acc2, (tuple, list)):
        acc2 = acc2[0]

    out = pl.pallas_call(
        functools.partial(_tc_head, G=G),
        out_shape=jax.ShapeDtypeStruct((G, C), jnp.float32),
    )(self2, acc2, batch.reshape(N, 1), Wfc, bfc.reshape(1, C))

    return out
